# baseline (speedup 1.0000x reference)
import jax
import jax.numpy as jnp
from jax.experimental import pallas as pl

_BB = 128


def _blend_body(y_ref, t_ref, x_ref, o_ref):
    m = (y_ref[...] == 0).astype(jnp.float32)
    t = t_ref[...]
    xv = x_ref[...]
    o_ref[...] = xv - (m * t) * (xv + 0.75)


def kernel(x, y, template):
    B, C, S, _ = x.shape
    F = C * S * S
    x2 = x.reshape(B, F)
    trow = jnp.tile(template.reshape(1, S * S), (1, C))
    out = pl.pallas_call(
        _blend_body,
        grid=(B // _BB,),
        in_specs=[
            pl.BlockSpec((_BB, 1), lambda i: (i, 0)),
            pl.BlockSpec((1, F), lambda i: (0, 0)),
            pl.BlockSpec((_BB, F), lambda i: (i, 0)),
        ],
        out_specs=pl.BlockSpec((_BB, F), lambda i: (i, 0)),
        out_shape=jax.ShapeDtypeStruct((B, F), x.dtype),
    )(y, trow, x2)
    return (out.reshape(x.shape), y)
